# baseline (speedup 1.0000x reference)
import functools

import jax
import jax.numpy as jnp
from jax import lax
from jax.experimental import pallas as pl
from jax.experimental.pallas import tpu as pltpu
from jax.experimental.pallas import tpu_sc as plsc

N_NODES = 10000
D = 128
N_EDGES = 320000
NC = 2
NS = 16
NW = NC * NS
K = 48
BLK = 42
NBLK = 5
CH = BLK * NBLK
E_PAD = NW * CH * K
N_PAD = 10240
SROWS = N_PAD // NS
WR = 40

_mesh = plsc.VectorSubcoreMesh(core_axis_name="c", subcore_axis_name="s")


@functools.partial(
    pl.kernel,
    out_type=jax.ShapeDtypeStruct((NC, N_PAD, D), jnp.float32),
    mesh=_mesh,
    scratch_types=[
        pltpu.VMEM((2, BLK, K), jnp.int32),
        pltpu.VMEM((BLK, K), jnp.float32),
        pltpu.VMEM((K, D), jnp.float32),
        pltpu.VMEM((K, D), jnp.float32),
        pltpu.VMEM((K, D), jnp.float32),
        pltpu.VMEM_SHARED((N_PAD, D), jnp.float32),
        pltpu.SemaphoreType.DMA,
        pltpu.SemaphoreType.DMA,
        pltpu.SemaphoreType.DMA,
        pltpu.SemaphoreType.DMA,
    ],
)
def _sc_aggregate(x_hbm, ed_hbm, ew_hbm, part_hbm,
                  ib, wb, r0, r1, r2, acc, sem_g, ss0, ss1, ss2):
    c = lax.axis_index("c")
    s = lax.axis_index("s")
    wid = c * NS + s
    bufs = (r0, r1, r2)
    ssems = (ss0, ss1, ss2)

    def _zrow(r, carry):
        for q in range(D // 16):
            r0[r, pl.ds(q * 16, 16)] = jnp.zeros((16,), jnp.float32)
        return carry
    lax.fori_loop(0, K, _zrow, 0)
    for i in range(SROWS // WR):
        pltpu.sync_copy(r0.at[pl.ds(0, WR)],
                        acc.at[pl.ds(s * SROWS + i * WR, WR)])
    plsc.subcore_barrier()

    def _scale(rows, l):
        def _grp(g, carry):
            wv = wb[l, pl.ds(g * 16, 16)]
            for lane in range(16):
                w = jnp.full((16,), wv[lane], dtype=jnp.float32)
                e = g * 16 + lane
                for q in range(D // 16):
                    sl = pl.ds(q * 16, 16)
                    rows[e, sl] = rows[e, sl] * w
            return carry
        lax.fori_loop(0, K // 16, _grp, 0)

    def _block(bb, carry):
        pltpu.sync_copy(ed_hbm.at[wid, bb], ib)
        pltpu.sync_copy(ew_hbm.at[wid, bb], wb)
        pltpu.async_copy(x_hbm.at[ib.at[0, 0]], r0, sem_g)

        def _triple(t, carry2):
            for r in range(3):
                j = 3 * t + r
                cur = bufs[r]
                nxt = bufs[(r + 1) % 3]
                pltpu.make_async_copy(
                    x_hbm.at[ib.at[0, j]], cur, sem_g).wait()
                if r < 2:
                    @pl.when(t > 0)
                    def _():
                        pltpu.make_async_copy(
                            nxt, acc.at[ib.at[1, j]], ssems[(r + 1) % 3]
                        ).wait()
                else:
                    pltpu.make_async_copy(
                        nxt, acc.at[ib.at[1, j]], ssems[(r + 1) % 3]).wait()
                if r < 2:
                    pltpu.async_copy(x_hbm.at[ib.at[0, j + 1]], nxt, sem_g)
                else:
                    @pl.when(t < BLK // 3 - 1)
                    def _():
                        pltpu.async_copy(
                            x_hbm.at[ib.at[0, j + 1]], nxt, sem_g)
                _scale(cur, j)
                pltpu.async_copy(cur, acc.at[ib.at[1, j]], ssems[r],
                                 add=True)
            return carry2
        lax.fori_loop(0, BLK // 3, _triple, 0)

        pltpu.make_async_copy(r1, acc.at[ib.at[1, 0]], ss1).wait()
        pltpu.make_async_copy(r2, acc.at[ib.at[1, 0]], ss2).wait()
        return carry
    lax.fori_loop(0, NBLK, _block, 0)
    plsc.subcore_barrier()

    for i in range(SROWS // WR):
        rr = s * SROWS + i * WR
        pltpu.sync_copy(acc.at[pl.ds(rr, WR)], r0.at[pl.ds(0, WR)])
        pltpu.sync_copy(r0.at[pl.ds(0, WR)], part_hbm.at[c, pl.ds(rr, WR)])


_RB = 1000


def _tc_body(p_ref, wt_ref, b_ref, o_ref):
    x = p_ref[0] + p_ref[1]
    o_ref[...] = (
        jnp.dot(x, wt_ref[...], preferred_element_type=jnp.float32) + b_ref[...]
    )


def _tc_linear(parts, wt, b2):
    return pl.pallas_call(
        _tc_body,
        out_shape=jax.ShapeDtypeStruct((N_NODES, D), jnp.float32),
        grid=(N_NODES // _RB,),
        in_specs=[
            pl.BlockSpec((NC, _RB, D), lambda i: (0, i, 0)),
            pl.BlockSpec((D, D), lambda i: (0, 0)),
            pl.BlockSpec((1, D), lambda i: (0, 0)),
        ],
        out_specs=pl.BlockSpec((_RB, D), lambda i: (i, 0)),
    )(parts, wt, b2)


@jax.jit
def _run(X, ed, ew4, wt, b2):
    parts = _sc_aggregate(X, ed, ew4)
    return _tc_linear(parts, wt, b2)


def kernel(X, edge_index, edge_weight, W, b):
    pad = E_PAD - N_EDGES
    pad_ids = jnp.arange(pad, dtype=jnp.int32)
    src = jnp.concatenate(
        [edge_index[0].astype(jnp.int32), pad_ids % N_NODES])
    dst = jnp.concatenate(
        [edge_index[1].astype(jnp.int32),
         N_NODES + pad_ids % (N_PAD - N_NODES)])
    ew4 = jnp.concatenate(
        [edge_weight, jnp.zeros((pad,), jnp.float32)]).reshape(NW, NBLK, BLK, K)
    ed = jnp.stack([src.reshape(NW, NBLK, BLK, K),
                    dst.reshape(NW, NBLK, BLK, K)], axis=2)
    return _run(X, ed, ew4, W.T, b.reshape(1, D))


# speedup vs baseline: 1.1066x; 1.1066x over previous
import functools

import jax
import jax.numpy as jnp
from jax import lax
from jax.experimental import pallas as pl
from jax.experimental.pallas import tpu as pltpu
from jax.experimental.pallas import tpu_sc as plsc

N_NODES = 10000
D = 128
N_EDGES = 320000
NC = 2
NS = 16
NW = NC * NS
K = 64
BLK = 18
NBLK = 9
CH = BLK * NBLK
E_PAD = NW * CH * K
N_PAD = 10240
SROWS = N_PAD // NS

_mesh = plsc.VectorSubcoreMesh(core_axis_name="c", subcore_axis_name="s")


@functools.partial(
    pl.kernel,
    out_type=jax.ShapeDtypeStruct((NC, N_PAD, D), jnp.float32),
    mesh=_mesh,
    scratch_types=[
        pltpu.VMEM((2, BLK, K), jnp.int32),
        pltpu.VMEM((BLK, K), jnp.float32),
        pltpu.VMEM((K, D), jnp.float32),
        pltpu.VMEM((K, D), jnp.float32),
        pltpu.VMEM((K, D), jnp.float32),
        pltpu.VMEM_SHARED((N_PAD, D), jnp.float32),
        pltpu.SemaphoreType.DMA,
        pltpu.SemaphoreType.DMA,
        pltpu.SemaphoreType.DMA,
        pltpu.SemaphoreType.DMA,
    ],
)
def _sc_aggregate(x_hbm, ed_hbm, ew_hbm, part_hbm,
                  ib, wb, r0, r1, r2, acc, sem_g, ss0, ss1, ss2):
    c = lax.axis_index("c")
    s = lax.axis_index("s")
    wid = c * NS + s
    bufs = (r0, r1, r2)
    ssems = (ss0, ss1, ss2)

    def _zrow(r, carry):
        for q in range(D // 16):
            r0[r, pl.ds(q * 16, 16)] = jnp.zeros((16,), jnp.float32)
        return carry
    lax.fori_loop(0, K, _zrow, 0)
    for i in range(SROWS // K):
        pltpu.sync_copy(r0, acc.at[pl.ds(s * SROWS + i * K, K)])
    plsc.subcore_barrier()

    def _scale(rows, l):
        def _grp(g, carry):
            wv = wb[l, pl.ds(g * 16, 16)]
            for lane in range(16):
                w = jnp.full((16,), wv[lane], dtype=jnp.float32)
                e = g * 16 + lane
                for q in range(D // 16):
                    sl = pl.ds(q * 16, 16)
                    rows[e, sl] = rows[e, sl] * w
            return carry
        lax.fori_loop(0, K // 16, _grp, 0)

    def _block(bb, carry):
        pltpu.sync_copy(ed_hbm.at[wid, bb], ib)
        pltpu.sync_copy(ew_hbm.at[wid, bb], wb)
        pltpu.async_copy(x_hbm.at[ib.at[0, 0]], r0, sem_g)

        def _triple(t, carry2):
            for r in range(3):
                j = 3 * t + r
                cur = bufs[r]
                nxt = bufs[(r + 1) % 3]
                pltpu.make_async_copy(
                    x_hbm.at[ib.at[0, j]], cur, sem_g).wait()
                if r < 2:
                    @pl.when(t > 0)
                    def _():
                        pltpu.make_async_copy(
                            nxt, acc.at[ib.at[1, j]], ssems[(r + 1) % 3]
                        ).wait()
                else:
                    pltpu.make_async_copy(
                        nxt, acc.at[ib.at[1, j]], ssems[(r + 1) % 3]).wait()
                if r < 2:
                    pltpu.async_copy(x_hbm.at[ib.at[0, j + 1]], nxt, sem_g)
                else:
                    @pl.when(t < BLK // 3 - 1)
                    def _():
                        pltpu.async_copy(
                            x_hbm.at[ib.at[0, j + 1]], nxt, sem_g)
                _scale(cur, j)
                pltpu.async_copy(cur, acc.at[ib.at[1, j]], ssems[r],
                                 add=True)
            return carry2
        lax.fori_loop(0, BLK // 3, _triple, 0)

        pltpu.make_async_copy(r1, acc.at[ib.at[1, 0]], ss1).wait()
        pltpu.make_async_copy(r2, acc.at[ib.at[1, 0]], ss2).wait()
        return carry
    lax.fori_loop(0, NBLK, _block, 0)
    plsc.subcore_barrier()

    for i in range(SROWS // K):
        rr = s * SROWS + i * K
        pltpu.sync_copy(acc.at[pl.ds(rr, K)], r0)
        pltpu.sync_copy(r0, part_hbm.at[c, pl.ds(rr, K)])


_RB = 1000


def _tc_body(p_ref, wt_ref, b_ref, o_ref):
    x = p_ref[0] + p_ref[1]
    o_ref[...] = (
        jnp.dot(x, wt_ref[...], preferred_element_type=jnp.float32) + b_ref[...]
    )


def _tc_linear(parts, wt, b2):
    return pl.pallas_call(
        _tc_body,
        out_shape=jax.ShapeDtypeStruct((N_NODES, D), jnp.float32),
        grid=(N_NODES // _RB,),
        in_specs=[
            pl.BlockSpec((NC, _RB, D), lambda i: (0, i, 0)),
            pl.BlockSpec((D, D), lambda i: (0, 0)),
            pl.BlockSpec((1, D), lambda i: (0, 0)),
        ],
        out_specs=pl.BlockSpec((_RB, D), lambda i: (i, 0)),
    )(parts, wt, b2)


@jax.jit
def _run(X, ed, ew4, wt, b2):
    parts = _sc_aggregate(X, ed, ew4)
    return _tc_linear(parts, wt, b2)


def kernel(X, edge_index, edge_weight, W, b):
    pad = E_PAD - N_EDGES
    pad_ids = jnp.arange(pad, dtype=jnp.int32)
    src = jnp.concatenate(
        [edge_index[0].astype(jnp.int32), pad_ids % N_NODES])
    dst = jnp.concatenate(
        [edge_index[1].astype(jnp.int32),
         N_NODES + pad_ids % (N_PAD - N_NODES)])
    ew4 = jnp.concatenate(
        [edge_weight, jnp.zeros((pad,), jnp.float32)]).reshape(NW, NBLK, BLK, K)
    ed = jnp.stack([src.reshape(NW, NBLK, BLK, K),
                    dst.reshape(NW, NBLK, BLK, K)], axis=2)
    return _run(X, ed, ew4, W.T, b.reshape(1, D))


# speedup vs baseline: 1.3588x; 1.2279x over previous
import functools

import jax
import jax.numpy as jnp
from jax import lax
from jax.experimental import pallas as pl
from jax.experimental.pallas import tpu as pltpu
from jax.experimental.pallas import tpu_sc as plsc

N_NODES = 10000
D = 128
N_EDGES = 320000
NC = 2
NS = 16
NW = NC * NS
K = 48
BLK = 24
NBLK = 9
CH = BLK * NBLK
E_PAD = NW * CH * K
N_PAD = 10240
SROWS = N_PAD // NS
WR = 40

_mesh = plsc.VectorSubcoreMesh(core_axis_name="c", subcore_axis_name="s")


@functools.partial(
    pl.kernel,
    out_type=jax.ShapeDtypeStruct((NC, N_PAD, D), jnp.float32),
    mesh=_mesh,
    scratch_types=[
        pltpu.VMEM((2, BLK, K), jnp.int32),
        pltpu.VMEM((BLK, K), jnp.float32),
        pltpu.VMEM((K, D), jnp.float32),
        pltpu.VMEM((K, D), jnp.float32),
        pltpu.VMEM((K, D), jnp.float32),
        pltpu.VMEM((K, D), jnp.float32),
        pltpu.VMEM_SHARED((N_PAD, D), jnp.float32),
        pltpu.SemaphoreType.DMA,
        pltpu.SemaphoreType.DMA,
        pltpu.SemaphoreType.DMA,
        pltpu.SemaphoreType.DMA,
        pltpu.SemaphoreType.DMA,
        pltpu.SemaphoreType.DMA,
        pltpu.SemaphoreType.DMA,
        pltpu.SemaphoreType.DMA,
    ],
)
def _sc_aggregate(x_hbm, ed_hbm, ew_hbm, part_hbm,
                  ib, wb, r0, r1, r2, r3, acc,
                  sg0, sg1, sg2, sg3, ss0, ss1, ss2, ss3):
    c = lax.axis_index("c")
    s = lax.axis_index("s")
    wid = c * NS + s
    bufs = (r0, r1, r2, r3)
    gsems = (sg0, sg1, sg2, sg3)
    ssems = (ss0, ss1, ss2, ss3)

    def _zrow(r, carry):
        for q in range(D // 16):
            r0[r, pl.ds(q * 16, 16)] = jnp.zeros((16,), jnp.float32)
        return carry
    lax.fori_loop(0, K, _zrow, 0)
    for i in range(SROWS // WR):
        pltpu.sync_copy(r0.at[pl.ds(0, WR)],
                        acc.at[pl.ds(s * SROWS + i * WR, WR)])
    plsc.subcore_barrier()

    def _scale(rows, l):
        def _grp(g, carry):
            wv = wb[l, pl.ds(g * 16, 16)]
            for lane in range(16):
                w = jnp.full((16,), wv[lane], dtype=jnp.float32)
                e = g * 16 + lane
                for q in range(D // 16):
                    sl = pl.ds(q * 16, 16)
                    rows[e, sl] = rows[e, sl] * w
            return carry
        lax.fori_loop(0, K // 16, _grp, 0)

    def _block(bb, carry):
        pltpu.sync_copy(ed_hbm.at[wid, bb], ib)
        pltpu.sync_copy(ew_hbm.at[wid, bb], wb)
        pltpu.async_copy(x_hbm.at[ib.at[0, 0]], r0, sg0)
        pltpu.async_copy(x_hbm.at[ib.at[0, 1]], r1, sg1)

        def _quad(t, carry2):
            for r in range(4):
                j = 4 * t + r
                cur = bufs[r]
                gbuf = bufs[(r + 2) % 4]
                pltpu.make_async_copy(
                    x_hbm.at[ib.at[0, j]], cur, gsems[r]).wait()
                if r < 2:
                    @pl.when(t > 0)
                    def _():
                        pltpu.make_async_copy(
                            gbuf, acc.at[ib.at[1, j]], ssems[(r + 2) % 4]
                        ).wait()
                else:
                    pltpu.make_async_copy(
                        gbuf, acc.at[ib.at[1, j]], ssems[(r + 2) % 4]).wait()
                if r < 2:
                    pltpu.async_copy(
                        x_hbm.at[ib.at[0, j + 2]], gbuf, gsems[(r + 2) % 4])
                else:
                    @pl.when(t < BLK // 4 - 1)
                    def _():
                        pltpu.async_copy(
                            x_hbm.at[ib.at[0, j + 2]], gbuf,
                            gsems[(r + 2) % 4])
                _scale(cur, j)
                pltpu.async_copy(cur, acc.at[ib.at[1, j]], ssems[r],
                                 add=True)
            return carry2
        lax.fori_loop(0, BLK // 4, _quad, 0)

        pltpu.make_async_copy(r2, acc.at[ib.at[1, 0]], ss2).wait()
        pltpu.make_async_copy(r3, acc.at[ib.at[1, 0]], ss3).wait()
        return carry
    lax.fori_loop(0, NBLK, _block, 0)
    plsc.subcore_barrier()

    for i in range(SROWS // WR):
        rr = s * SROWS + i * WR
        pltpu.sync_copy(acc.at[pl.ds(rr, WR)], r0.at[pl.ds(0, WR)])
        pltpu.sync_copy(r0.at[pl.ds(0, WR)], part_hbm.at[c, pl.ds(rr, WR)])


_RB = 1000


def _tc_body(p_ref, wt_ref, b_ref, o_ref):
    x = p_ref[0] + p_ref[1]
    o_ref[...] = (
        jnp.dot(x, wt_ref[...], preferred_element_type=jnp.float32) + b_ref[...]
    )


def _tc_linear(parts, wt, b2):
    return pl.pallas_call(
        _tc_body,
        out_shape=jax.ShapeDtypeStruct((N_NODES, D), jnp.float32),
        grid=(N_NODES // _RB,),
        in_specs=[
            pl.BlockSpec((NC, _RB, D), lambda i: (0, i, 0)),
            pl.BlockSpec((D, D), lambda i: (0, 0)),
            pl.BlockSpec((1, D), lambda i: (0, 0)),
        ],
        out_specs=pl.BlockSpec((_RB, D), lambda i: (i, 0)),
    )(parts, wt, b2)


@jax.jit
def _run(X, ed, ew4, wt, b2):
    parts = _sc_aggregate(X, ed, ew4)
    return _tc_linear(parts, wt, b2)


def kernel(X, edge_index, edge_weight, W, b):
    pad = E_PAD - N_EDGES
    pad_ids = jnp.arange(pad, dtype=jnp.int32)
    src = jnp.concatenate(
        [edge_index[0].astype(jnp.int32), pad_ids % N_NODES])
    dst = jnp.concatenate(
        [edge_index[1].astype(jnp.int32),
         N_NODES + pad_ids % (N_PAD - N_NODES)])
    ew4 = jnp.concatenate(
        [edge_weight, jnp.zeros((pad,), jnp.float32)]).reshape(NW, NBLK, BLK, K)
    ed = jnp.stack([src.reshape(NW, NBLK, BLK, K),
                    dst.reshape(NW, NBLK, BLK, K)], axis=2)
    return _run(X, ed, ew4, W.T, b.reshape(1, D))


# speedup vs baseline: 1.5180x; 1.1172x over previous
import functools

import jax
import jax.numpy as jnp
from jax import lax
from jax.experimental import pallas as pl
from jax.experimental.pallas import tpu as pltpu
from jax.experimental.pallas import tpu_sc as plsc

N_NODES = 10000
D = 128
N_EDGES = 320000
NC = 2
NS = 16
NW = NC * NS
K = 40
BLK = 25
NBLK = 10
CH = BLK * NBLK
N_PAD = 10240
SROWS = N_PAD // NS

_mesh = plsc.VectorSubcoreMesh(core_axis_name="c", subcore_axis_name="s")


@functools.partial(
    pl.kernel,
    out_type=jax.ShapeDtypeStruct((NC, N_PAD, D), jnp.float32),
    mesh=_mesh,
    scratch_types=[
        pltpu.VMEM((BLK, K), jnp.int32),
        pltpu.VMEM((BLK, K), jnp.int32),
        pltpu.VMEM((BLK, K), jnp.float32),
        pltpu.VMEM((K, D), jnp.float32),
        pltpu.VMEM((K, D), jnp.float32),
        pltpu.VMEM((K, D), jnp.float32),
        pltpu.VMEM((K, D), jnp.float32),
        pltpu.VMEM((K, D), jnp.float32),
        pltpu.VMEM_SHARED((N_PAD, D), jnp.float32),
        pltpu.SemaphoreType.DMA,
        pltpu.SemaphoreType.DMA,
        pltpu.SemaphoreType.DMA,
        pltpu.SemaphoreType.DMA,
        pltpu.SemaphoreType.DMA,
        pltpu.SemaphoreType.DMA,
        pltpu.SemaphoreType.DMA,
        pltpu.SemaphoreType.DMA,
        pltpu.SemaphoreType.DMA,
        pltpu.SemaphoreType.DMA,
    ],
)
def _sc_aggregate(x_hbm, ed_hbm, ew_hbm, part_hbm,
                  ibs, ibd, wb, r0, r1, r2, r3, r4, acc,
                  sg0, sg1, sg2, sg3, sg4, ss0, ss1, ss2, ss3, ss4):
    c = lax.axis_index("c")
    s = lax.axis_index("s")
    wid = c * NS + s
    bufs = (r0, r1, r2, r3, r4)
    gsems = (sg0, sg1, sg2, sg3, sg4)
    ssems = (ss0, ss1, ss2, ss3, ss4)

    def _zrow(r, carry):
        for q in range(D // 16):
            r0[r, pl.ds(q * 16, 16)] = jnp.zeros((16,), jnp.float32)
        return carry
    lax.fori_loop(0, K, _zrow, 0)
    for i in range(SROWS // K):
        pltpu.sync_copy(r0, acc.at[pl.ds(s * SROWS + i * K, K)])
    plsc.subcore_barrier()

    def _scale(rows, l):
        def _grp(g, carry):
            wv = wb[l, pl.ds(g * 16, 16)]
            for lane in range(16):
                w = jnp.full((16,), wv[lane], dtype=jnp.float32)
                e = g * 16 + lane
                for q in range(D // 16):
                    sl = pl.ds(q * 16, 16)
                    rows[e, sl] = rows[e, sl] * w
            return carry
        lax.fori_loop(0, 2, _grp, 0)
        wv = wb[l, pl.ds(24, 16)]
        for lane in range(8, 16):
            w = jnp.full((16,), wv[lane], dtype=jnp.float32)
            e = 24 + lane
            for q in range(D // 16):
                sl = pl.ds(q * 16, 16)
                rows[e, sl] = rows[e, sl] * w

    def _block(bb, carry):
        pltpu.sync_copy(ed_hbm.at[0, wid, bb], ibs)
        pltpu.sync_copy(ed_hbm.at[1, wid, bb], ibd)
        pltpu.sync_copy(ew_hbm.at[wid, bb], wb)
        pltpu.async_copy(x_hbm.at[ibs.at[0]], r0, sg0)
        pltpu.async_copy(x_hbm.at[ibs.at[1]], r1, sg1)
        pltpu.async_copy(x_hbm.at[ibs.at[2]], r2, sg2)

        def _quint(t, carry2):
            for r in range(5):
                j = 5 * t + r
                cur = bufs[r]
                gbuf = bufs[(r + 3) % 5]
                pltpu.make_async_copy(
                    x_hbm.at[ibs.at[j]], cur, gsems[r]).wait()
                if r < 2:
                    @pl.when(t > 0)
                    def _():
                        pltpu.make_async_copy(
                            gbuf, acc.at[ibd.at[j]], ssems[(r + 3) % 5]
                        ).wait()
                else:
                    pltpu.make_async_copy(
                        gbuf, acc.at[ibd.at[j]], ssems[(r + 3) % 5]).wait()
                if r < 2:
                    pltpu.async_copy(
                        x_hbm.at[ibs.at[j + 3]], gbuf, gsems[(r + 3) % 5])
                else:
                    @pl.when(t < BLK // 5 - 1)
                    def _():
                        pltpu.async_copy(
                            x_hbm.at[ibs.at[j + 3]], gbuf,
                            gsems[(r + 3) % 5])
                _scale(cur, j)
                pltpu.async_copy(cur, acc.at[ibd.at[j]], ssems[r],
                                 add=True)
            return carry2
        lax.fori_loop(0, BLK // 5, _quint, 0)

        pltpu.make_async_copy(r3, acc.at[ibd.at[0]], ss3).wait()
        pltpu.make_async_copy(r4, acc.at[ibd.at[0]], ss4).wait()
        return carry
    lax.fori_loop(0, NBLK, _block, 0)
    plsc.subcore_barrier()

    for i in range(SROWS // K):
        rr = s * SROWS + i * K
        pltpu.sync_copy(acc.at[pl.ds(rr, K)], r0)
        pltpu.sync_copy(r0, part_hbm.at[c, pl.ds(rr, K)])


_RB = 1000


def _tc_body(p_ref, wt_ref, b_ref, o_ref):
    x = p_ref[0] + p_ref[1]
    o_ref[...] = (
        jnp.dot(x, wt_ref[...], preferred_element_type=jnp.float32) + b_ref[...]
    )


def _tc_linear(parts, wt, b2):
    return pl.pallas_call(
        _tc_body,
        out_shape=jax.ShapeDtypeStruct((N_NODES, D), jnp.float32),
        grid=(N_NODES // _RB,),
        in_specs=[
            pl.BlockSpec((NC, _RB, D), lambda i: (0, i, 0)),
            pl.BlockSpec((D, D), lambda i: (0, 0)),
            pl.BlockSpec((1, D), lambda i: (0, 0)),
        ],
        out_specs=pl.BlockSpec((_RB, D), lambda i: (i, 0)),
    )(parts, wt, b2)


@jax.jit
def _run(X, ed, ew4, wt, b2):
    parts = _sc_aggregate(X, ed, ew4)
    return _tc_linear(parts, wt, b2)


def kernel(X, edge_index, edge_weight, W, b):
    ed = edge_index.astype(jnp.int32).reshape(2, NW, NBLK, BLK, K)
    ew4 = edge_weight.reshape(NW, NBLK, BLK, K)
    return _run(X, ed, ew4, W.T, b.reshape(1, D))


# speedup vs baseline: 1.6611x; 1.0942x over previous
import functools

import jax
import jax.numpy as jnp
from jax import lax
from jax.experimental import pallas as pl
from jax.experimental.pallas import tpu as pltpu
from jax.experimental.pallas import tpu_sc as plsc

N_NODES = 10000
D = 128
N_EDGES = 320000
NC = 2
NS = 16
NW = NC * NS
K = 40
BLK = 25
NBLK = 10
CH = BLK * NBLK
N_PAD = 10240
SROWS = N_PAD // NS

_mesh = plsc.VectorSubcoreMesh(core_axis_name="c", subcore_axis_name="s")


@functools.partial(
    pl.kernel,
    out_type=jax.ShapeDtypeStruct((NC, N_PAD, D), jnp.float32),
    mesh=_mesh,
    scratch_types=[
        pltpu.VMEM((BLK, K), jnp.int32),
        pltpu.VMEM((BLK, K), jnp.int32),
        pltpu.VMEM((BLK, K), jnp.float32),
        pltpu.VMEM((K, D), jnp.float32),
        pltpu.VMEM((K, D), jnp.float32),
        pltpu.VMEM((K, D), jnp.float32),
        pltpu.VMEM((K, D), jnp.float32),
        pltpu.VMEM((K, D), jnp.float32),
        pltpu.VMEM_SHARED((N_PAD, D), jnp.float32),
        pltpu.SemaphoreType.DMA,
        pltpu.SemaphoreType.DMA,
        pltpu.SemaphoreType.DMA,
        pltpu.SemaphoreType.DMA,
        pltpu.SemaphoreType.DMA,
        pltpu.SemaphoreType.DMA,
        pltpu.SemaphoreType.DMA,
        pltpu.SemaphoreType.DMA,
        pltpu.SemaphoreType.DMA,
        pltpu.SemaphoreType.DMA,
    ],
)
def _sc_aggregate(x_hbm, ed_hbm, ew_hbm, part_hbm,
                  ibs, ibd, wb, r0, r1, r2, r3, r4, acc,
                  sg0, sg1, sg2, sg3, sg4, ss0, ss1, ss2, ss3, ss4):
    c = lax.axis_index("c")
    s = lax.axis_index("s")
    wid = c * NS + s
    bufs = (r0, r1, r2, r3, r4)
    gsems = (sg0, sg1, sg2, sg3, sg4)
    ssems = (ss0, ss1, ss2, ss3, ss4)

    def _zrow(r, carry):
        for q in range(D // 16):
            r0[r, pl.ds(q * 16, 16)] = jnp.zeros((16,), jnp.float32)
        return carry
    lax.fori_loop(0, K, _zrow, 0)
    for i in range(SROWS // K):
        pltpu.async_copy(r0, acc.at[pl.ds(s * SROWS + i * K, K)], sg0)
    for i in range(SROWS // K):
        pltpu.make_async_copy(r0, acc.at[pl.ds(s * SROWS, K)], sg0).wait()
    plsc.subcore_barrier()

    def _scale(rows, l):
        def _grp(g, carry):
            wv = wb[l, pl.ds(g * 16, 16)]
            for lane in range(16):
                w = jnp.full((16,), wv[lane], dtype=jnp.float32)
                e = g * 16 + lane
                for q in range(D // 16):
                    sl = pl.ds(q * 16, 16)
                    rows[e, sl] = rows[e, sl] * w
            return carry
        lax.fori_loop(0, 2, _grp, 0)
        wv = wb[l, pl.ds(24, 16)]
        for lane in range(8, 16):
            w = jnp.full((16,), wv[lane], dtype=jnp.float32)
            e = 24 + lane
            for q in range(D // 16):
                sl = pl.ds(q * 16, 16)
                rows[e, sl] = rows[e, sl] * w

    def _block(bb, carry):
        pltpu.async_copy(ed_hbm.at[0, wid, bb], ibs, sg0)
        pltpu.async_copy(ed_hbm.at[1, wid, bb], ibd, sg1)
        pltpu.async_copy(ew_hbm.at[wid, bb], wb, sg2)
        pltpu.make_async_copy(ed_hbm.at[0, wid, bb], ibs, sg0).wait()
        pltpu.make_async_copy(ed_hbm.at[1, wid, bb], ibd, sg1).wait()
        pltpu.make_async_copy(ew_hbm.at[wid, bb], wb, sg2).wait()
        pltpu.async_copy(x_hbm.at[ibs.at[0]], r0, sg0)
        pltpu.async_copy(x_hbm.at[ibs.at[1]], r1, sg1)
        pltpu.async_copy(x_hbm.at[ibs.at[2]], r2, sg2)

        def _quint(t, carry2):
            for r in range(5):
                j = 5 * t + r
                cur = bufs[r]
                gbuf = bufs[(r + 3) % 5]
                pltpu.make_async_copy(
                    x_hbm.at[ibs.at[j]], cur, gsems[r]).wait()
                if r < 2:
                    @pl.when(t > 0)
                    def _():
                        pltpu.make_async_copy(
                            gbuf, acc.at[ibd.at[j]], ssems[(r + 3) % 5]
                        ).wait()
                else:
                    pltpu.make_async_copy(
                        gbuf, acc.at[ibd.at[j]], ssems[(r + 3) % 5]).wait()
                if r < 2:
                    pltpu.async_copy(
                        x_hbm.at[ibs.at[j + 3]], gbuf, gsems[(r + 3) % 5])
                else:
                    @pl.when(t < BLK // 5 - 1)
                    def _():
                        pltpu.async_copy(
                            x_hbm.at[ibs.at[j + 3]], gbuf,
                            gsems[(r + 3) % 5])
                _scale(cur, j)
                pltpu.async_copy(cur, acc.at[ibd.at[j]], ssems[r],
                                 add=True)
            return carry2
        lax.fori_loop(0, BLK // 5, _quint, 0)

        pltpu.make_async_copy(r3, acc.at[ibd.at[0]], ss3).wait()
        pltpu.make_async_copy(r4, acc.at[ibd.at[0]], ss4).wait()
        return carry
    lax.fori_loop(0, NBLK, _block, 0)
    plsc.subcore_barrier()

    for i in range(SROWS // K):
        rr = s * SROWS + i * K
        b = bufs[i % 5]
        if i >= 5:
            pltpu.make_async_copy(b, part_hbm.at[c, pl.ds(rr, K)],
                                  ssems[i % 5]).wait()
        pltpu.sync_copy(acc.at[pl.ds(rr, K)], b)
        pltpu.async_copy(b, part_hbm.at[c, pl.ds(rr, K)], ssems[i % 5])
    for i in range(5):
        pltpu.make_async_copy(bufs[i], part_hbm.at[c, pl.ds(s * SROWS, K)],
                              ssems[i]).wait()


_RB = 1000


def _tc_body(p_ref, wt_ref, b_ref, o_ref):
    x = p_ref[0] + p_ref[1]
    o_ref[...] = (
        jnp.dot(x, wt_ref[...], preferred_element_type=jnp.float32) + b_ref[...]
    )


def _tc_linear(parts, wt, b2):
    return pl.pallas_call(
        _tc_body,
        out_shape=jax.ShapeDtypeStruct((N_NODES, D), jnp.float32),
        grid=(N_NODES // _RB,),
        in_specs=[
            pl.BlockSpec((NC, _RB, D), lambda i: (0, i, 0)),
            pl.BlockSpec((D, D), lambda i: (0, 0)),
            pl.BlockSpec((1, D), lambda i: (0, 0)),
        ],
        out_specs=pl.BlockSpec((_RB, D), lambda i: (i, 0)),
    )(parts, wt, b2)


@jax.jit
def _run(X, ed, ew4, wt, b2):
    parts = _sc_aggregate(X, ed, ew4)
    return _tc_linear(parts, wt, b2)


def kernel(X, edge_index, edge_weight, W, b):
    ed = edge_index.astype(jnp.int32).reshape(2, NW, NBLK, BLK, K)
    ew4 = edge_weight.reshape(NW, NBLK, BLK, K)
    return _run(X, ed, ew4, W.T, b.reshape(1, D))
